# baseline (speedup 1.0000x reference)
import functools

import jax
import jax.numpy as jnp
from jax import lax
from jax.experimental import pallas as pl
from jax.experimental.pallas import tpu as pltpu
from jax.experimental.pallas import tpu_sc as plsc

B = 16384
D = 64
NC = 2
NS = 16
NW = NC * NS
B_PER_W = B // NW
CH = 32
NCH = B_PER_W // CH
GRP = 125000


_UNROLL = 256
_N_FIRE = B_PER_W // _UNROLL


def _sc_gather_body(table_hbm, idx_hbm, out_hbm, idx_v, sem):
    wid = lax.axis_index("s") * NC + lax.axis_index("c")
    base = wid * B_PER_W
    pltpu.sync_copy(idx_hbm.at[pl.ds(base, B_PER_W)], idx_v)

    def fire(h, _):
        off = h * _UNROLL
        for g in range(_UNROLL // 16):
            vec = idx_v[pl.ds(off + 16 * g, 16)]
            for j in range(16):
                pltpu.async_copy(
                    table_hbm.at[pl.ds(vec[j], 1)],
                    out_hbm.at[pl.ds(base + off + 16 * g + j, 1)],
                    sem,
                )
        return ()

    lax.fori_loop(0, _N_FIRE, fire, (), unroll=False)
    pltpu.make_async_copy(
        table_hbm.at[pl.ds(0, B_PER_W)],
        out_hbm.at[pl.ds(base, B_PER_W)],
        sem,
    ).wait()


@functools.cache
def _sc_gather():
    return pl.kernel(
        _sc_gather_body,
        out_type=jax.ShapeDtypeStruct((B, D), jnp.float32),
        mesh=plsc.VectorSubcoreMesh(core_axis_name="c", subcore_axis_name="s"),
        scratch_types=[
            pltpu.VMEM((B_PER_W,), jnp.int32),
            pltpu.SemaphoreType.DMA,
        ],
        compiler_params=pltpu.CompilerParams(needs_layout_passes=False),
    )


def _tc_loss_body(x_ref, g_ref, out_ref):
    i = pl.program_id(0)
    x = x_ref[...]
    g = g_ref[...]
    m = jnp.max(x, axis=1, keepdims=True)
    e = jnp.exp(x - m)
    z = jnp.sum(e, axis=1, keepdims=True)
    logsm = x - m - jnp.log(z)
    part = -jnp.sum(logsm * g, keepdims=True) * (1.0 / B)

    @pl.when(i == 0)
    def _init():
        out_ref[...] = part

    @pl.when(i != 0)
    def _acc():
        out_ref[...] += part


_N_BLK = 8
_BLK = B // _N_BLK

_tc_loss = pl.pallas_call(
    _tc_loss_body,
    grid=(_N_BLK,),
    in_specs=[
        pl.BlockSpec((_BLK, D), lambda i: (i, 0)),
        pl.BlockSpec((_BLK, D), lambda i: (i, 0)),
    ],
    out_specs=pl.BlockSpec((1, 1), lambda i: (0, 0)),
    out_shape=jax.ShapeDtypeStruct((1, 1), jnp.float32),
)


def kernel(outputs, index, confidence):
    idx = index.astype(jnp.int32)
    gathered = _sc_gather()(confidence, idx)
    loss = _tc_loss(outputs, gathered)
    return loss[0, 0]


# speedup vs baseline: 1.6529x; 1.6529x over previous
import functools

import jax
import jax.numpy as jnp
from jax import lax
from jax.experimental import pallas as pl
from jax.experimental.pallas import tpu as pltpu
from jax.experimental.pallas import tpu_sc as plsc

B = 16384
D = 64
NC = 2
NS = 16
NW = NC * NS
B_PER_W = B // NW
CH = 32
NCH = B_PER_W // CH
GRP = 125000


_UNROLL = 256
_N_FIRE = B_PER_W // _UNROLL


def _sc_gather_body(table_hbm, idx_hbm, out_hbm, idx_v, rows_v, sem):
    wid = lax.axis_index("s") * NC + lax.axis_index("c")
    base = wid * B_PER_W
    pltpu.sync_copy(idx_hbm.at[pl.ds(base, B_PER_W)], idx_v)

    def fire(h, _):
        off = h * _UNROLL
        for g in range(_UNROLL // 16):
            vec = idx_v[pl.ds(off + 16 * g, 16)]
            for j in range(16):
                i = off + 16 * g + j
                pltpu.async_copy(
                    table_hbm.at[pl.ds(vec[j], 1)],
                    rows_v.at[pl.ds(i, 1)],
                    sem,
                )
        return ()

    lax.fori_loop(0, _N_FIRE, fire, (), unroll=False)
    pltpu.make_async_copy(
        table_hbm.at[pl.ds(0, B_PER_W)],
        rows_v,
        sem,
    ).wait()
    pltpu.sync_copy(rows_v, out_hbm.at[pl.ds(base, B_PER_W)])


@functools.cache
def _sc_gather():
    return pl.kernel(
        _sc_gather_body,
        out_type=jax.ShapeDtypeStruct((B, D), jnp.float32),
        mesh=plsc.VectorSubcoreMesh(core_axis_name="c", subcore_axis_name="s"),
        scratch_types=[
            pltpu.VMEM((B_PER_W,), jnp.int32),
            pltpu.VMEM((B_PER_W, D), jnp.float32),
            pltpu.SemaphoreType.DMA,
        ],
        compiler_params=pltpu.CompilerParams(needs_layout_passes=False),
    )


def _tc_loss_body(x_ref, g_ref, out_ref):
    i = pl.program_id(0)
    x = x_ref[...]
    g = g_ref[...]
    m = jnp.max(x, axis=1, keepdims=True)
    e = jnp.exp(x - m)
    z = jnp.sum(e, axis=1, keepdims=True)
    logsm = x - m - jnp.log(z)
    part = -jnp.sum(logsm * g, keepdims=True) * (1.0 / B)

    @pl.when(i == 0)
    def _init():
        out_ref[...] = part

    @pl.when(i != 0)
    def _acc():
        out_ref[...] += part


_N_BLK = 8
_BLK = B // _N_BLK

_tc_loss = pl.pallas_call(
    _tc_loss_body,
    grid=(_N_BLK,),
    in_specs=[
        pl.BlockSpec((_BLK, D), lambda i: (i, 0)),
        pl.BlockSpec((_BLK, D), lambda i: (i, 0)),
    ],
    out_specs=pl.BlockSpec((1, 1), lambda i: (0, 0)),
    out_shape=jax.ShapeDtypeStruct((1, 1), jnp.float32),
)


def kernel(outputs, index, confidence):
    idx = index.astype(jnp.int32)
    gathered = _sc_gather()(confidence, idx)
    loss = _tc_loss(outputs, gathered)
    return loss[0, 0]


# speedup vs baseline: 1.6564x; 1.0021x over previous
import functools

import jax
import jax.numpy as jnp
from jax import lax
from jax.experimental import pallas as pl
from jax.experimental.pallas import tpu as pltpu
from jax.experimental.pallas import tpu_sc as plsc

B = 16384
D = 64
NC = 2
NS = 16
NW = NC * NS
B_PER_W = B // NW
CH = 32
NCH = B_PER_W // CH
GRP = 125000


_UNROLL = 256
_N_FIRE = B_PER_W // _UNROLL


_NSEM = 8


def _sc_gather_body(table_hbm, idx_hbm, out_hbm, idx_v, rows_v, sems):
    wid = lax.axis_index("s") * NC + lax.axis_index("c")
    base = wid * B_PER_W
    pltpu.sync_copy(idx_hbm.at[pl.ds(base, B_PER_W)], idx_v)

    def fire(h, _):
        off = h * _UNROLL
        for g in range(_UNROLL // 16):
            vec = idx_v[pl.ds(off + 16 * g, 16)]
            for j in range(16):
                i = off + 16 * g + j
                pltpu.async_copy(
                    table_hbm.at[pl.ds(vec[j], 1)],
                    rows_v.at[pl.ds(i, 1)],
                    sems.at[(16 * g + j) % _NSEM],
                )
        return ()

    lax.fori_loop(0, _N_FIRE, fire, (), unroll=False)
    per_sem = B_PER_W // _NSEM
    for k in range(_NSEM):
        pltpu.make_async_copy(
            table_hbm.at[pl.ds(0, per_sem)],
            rows_v.at[pl.ds(0, per_sem)],
            sems.at[k],
        ).wait()
    pltpu.sync_copy(rows_v, out_hbm.at[pl.ds(base, B_PER_W)])


@functools.cache
def _sc_gather():
    return pl.kernel(
        _sc_gather_body,
        out_type=jax.ShapeDtypeStruct((B, D), jnp.float32),
        mesh=plsc.VectorSubcoreMesh(core_axis_name="c", subcore_axis_name="s"),
        scratch_types=[
            pltpu.VMEM((B_PER_W,), jnp.int32),
            pltpu.VMEM((B_PER_W, D), jnp.float32),
            pltpu.SemaphoreType.DMA((_NSEM,)),
        ],
        compiler_params=pltpu.CompilerParams(needs_layout_passes=False),
    )


def _tc_loss_body(x_ref, g_ref, out_ref):
    i = pl.program_id(0)
    x = x_ref[...]
    g = g_ref[...]
    m = jnp.max(x, axis=1, keepdims=True)
    e = jnp.exp(x - m)
    z = jnp.sum(e, axis=1, keepdims=True)
    logsm = x - m - jnp.log(z)
    part = -jnp.sum(logsm * g, keepdims=True) * (1.0 / B)

    @pl.when(i == 0)
    def _init():
        out_ref[...] = part

    @pl.when(i != 0)
    def _acc():
        out_ref[...] += part


_N_BLK = 8
_BLK = B // _N_BLK

_tc_loss = pl.pallas_call(
    _tc_loss_body,
    grid=(_N_BLK,),
    in_specs=[
        pl.BlockSpec((_BLK, D), lambda i: (i, 0)),
        pl.BlockSpec((_BLK, D), lambda i: (i, 0)),
    ],
    out_specs=pl.BlockSpec((1, 1), lambda i: (0, 0)),
    out_shape=jax.ShapeDtypeStruct((1, 1), jnp.float32),
)


def kernel(outputs, index, confidence):
    idx = index.astype(jnp.int32)
    gathered = _sc_gather()(confidence, idx)
    loss = _tc_loss(outputs, gathered)
    return loss[0, 0]
